# baseline (speedup 1.0000x reference)
import functools

import jax
import jax.numpy as jnp
from jax import lax
from jax.experimental import pallas as pl
from jax.experimental.pallas import tpu as pltpu
from jax.experimental.pallas import tpu_sc as plsc

N_NODES = 10000
E = 320000
COORD_DIM = 128
D_EDGE = 16
H = 512
OUT_DIM = 3


@functools.cache
def _make_sc_gather(ec):
    info = plsc.get_sparse_core_info()
    nw = info.num_cores * info.num_subcores
    per_w = ec // nw
    ch = 400
    n_ch = per_w // ch
    mesh = plsc.VectorSubcoreMesh(core_axis_name="c", subcore_axis_name="s")

    @functools.partial(
        pl.kernel,
        mesh=mesh,
        out_type=jax.ShapeDtypeStruct((ec, 2 * COORD_DIM), jnp.float32),
        scratch_types=[
            pltpu.VMEM((ch,), jnp.int32),
            pltpu.VMEM((ch, COORD_DIM), jnp.float32),
            pltpu.VMEM((ch,), jnp.int32),
            pltpu.VMEM((ch, COORD_DIM), jnp.float32),
            pltpu.SemaphoreType.DMA,
            pltpu.SemaphoreType.DMA,
        ],
    )
    def sc_gather(coord_hbm, src_hbm, dst_hbm, x_hbm,
                  idxa_v, rowsa_v, idxb_v, rowsb_v, sema, semb):
        wid = lax.axis_index("s") * info.num_cores + lax.axis_index("c")
        base = wid * per_w

        def body(c, carry):
            off = base + c * ch
            pltpu.sync_copy(src_hbm.at[pl.ds(off, ch)], idxa_v)
            pltpu.sync_copy(dst_hbm.at[pl.ds(off, ch)], idxb_v)
            ga = pltpu.async_copy(coord_hbm.at[idxa_v], rowsa_v, sema)
            gb = pltpu.async_copy(coord_hbm.at[idxb_v], rowsb_v, semb)
            ga.wait()
            wa = pltpu.async_copy(
                rowsa_v, x_hbm.at[pl.ds(off, ch), pl.ds(0, COORD_DIM)], sema)
            gb.wait()
            wb = pltpu.async_copy(
                rowsb_v, x_hbm.at[pl.ds(off, ch), pl.ds(COORD_DIM, COORD_DIM)],
                semb)
            wa.wait()
            wb.wait()
            return carry

        lax.fori_loop(0, n_ch, body, 0)

    return sc_gather


def _mlp_body(x, f, nf, w0ab, w0c, b0, w1, b1, w2, b2, w3, b3, out):
    bf = jnp.bfloat16
    dot = functools.partial(jnp.dot, preferred_element_type=jnp.float32)
    h = dot(x[...].astype(bf), w0ab[...]) + dot(f[...], w0c[...])
    h = jnp.maximum(h + b0[...], 0.0).astype(bf)
    h = jnp.maximum(dot(h, w1[...]) + b1[...], 0.0).astype(bf)
    h = jnp.maximum(dot(h, w2[...]) + b2[...], 0.0).astype(bf)
    out[...] = (dot(h, w3[...]) + b3[...]) * nf[...]


def _mlp_call(x, feat, nf, w0ab, w0c, b0, w1, b1, w2, b2, w3, b3):
    ec = x.shape[0]
    blk = 2560
    grid = (ec // blk,)

    def row_spec(d):
        return pl.BlockSpec((blk, d), lambda i: (i, 0))

    def full_spec(shape):
        return pl.BlockSpec(shape, lambda i: (0,) * len(shape))

    return pl.pallas_call(
        _mlp_body,
        grid=grid,
        in_specs=[
            row_spec(2 * COORD_DIM),
            row_spec(D_EDGE),
            row_spec(1),
            full_spec(w0ab.shape),
            full_spec(w0c.shape),
            full_spec(b0.shape),
            full_spec(w1.shape),
            full_spec(b1.shape),
            full_spec(w2.shape),
            full_spec(b2.shape),
            full_spec(w3.shape),
            full_spec(b3.shape),
        ],
        out_specs=pl.BlockSpec((blk, OUT_DIM), lambda i: (i, 0)),
        out_shape=jax.ShapeDtypeStruct((ec, OUT_DIM), jnp.float32),
    )(x, feat, nf, w0ab, w0c, b0, w1, b1, w2, b2, w3, b3)


N_CHUNKS = 5


def kernel(coordinates, feature_array, non_fictitious, src, dst,
           W0, b0, W1, b1, W2, b2, W3, b3):
    bf = jnp.bfloat16
    w0ab = W0[:2 * COORD_DIM].astype(bf)
    w0c = W0[2 * COORD_DIM:].astype(bf)
    feat = feature_array.astype(bf)
    nf = non_fictitious.reshape(E, 1)
    weights = (w0ab, w0c, b0.reshape(1, H),
               W1.astype(bf), b1.reshape(1, H),
               W2.astype(bf), b2.reshape(1, H),
               W3.astype(bf), b3.reshape(1, OUT_DIM))
    ec = E // N_CHUNKS
    gather = _make_sc_gather(ec)
    outs = []
    for k in range(N_CHUNKS):
        sl = slice(k * ec, (k + 1) * ec)
        x = gather(coordinates, src[sl], dst[sl])
        outs.append(_mlp_call(x, feat[sl], nf[sl], *weights))
    return jnp.concatenate(outs, axis=0)


# speedup vs baseline: 1.1771x; 1.1771x over previous
import functools

import jax
import jax.numpy as jnp
from jax import lax
from jax.experimental import pallas as pl
from jax.experimental.pallas import tpu as pltpu
from jax.experimental.pallas import tpu_sc as plsc

N_NODES = 10000
E = 320000
COORD_DIM = 128
D_EDGE = 16
H = 512
OUT_DIM = 3


@functools.cache
def _make_sc_gather(ec):
    info = plsc.get_sparse_core_info()
    nw = info.num_cores * info.num_subcores
    per_w = ec // nw
    ch = 400
    n_ch = per_w // ch
    mesh = plsc.VectorSubcoreMesh(core_axis_name="c", subcore_axis_name="s")

    @functools.partial(
        pl.kernel,
        mesh=mesh,
        out_type=jax.ShapeDtypeStruct((ec, 2 * COORD_DIM), jnp.float32),
        scratch_types=[
            pltpu.VMEM((ch,), jnp.int32),
            pltpu.VMEM((ch, COORD_DIM), jnp.float32),
            pltpu.VMEM((ch,), jnp.int32),
            pltpu.VMEM((ch, COORD_DIM), jnp.float32),
            pltpu.SemaphoreType.DMA,
            pltpu.SemaphoreType.DMA,
        ],
    )
    def sc_gather(coord_hbm, src_hbm, dst_hbm, x_hbm,
                  idxa_v, rowsa_v, idxb_v, rowsb_v, sema, semb):
        wid = lax.axis_index("s") * info.num_cores + lax.axis_index("c")
        base = wid * per_w

        def body(c, carry):
            off = base + c * ch
            pltpu.sync_copy(src_hbm.at[pl.ds(off, ch)], idxa_v)
            pltpu.sync_copy(dst_hbm.at[pl.ds(off, ch)], idxb_v)
            ga = pltpu.async_copy(coord_hbm.at[idxa_v], rowsa_v, sema)
            gb = pltpu.async_copy(coord_hbm.at[idxb_v], rowsb_v, semb)
            ga.wait()
            wa = pltpu.async_copy(
                rowsa_v, x_hbm.at[pl.ds(off, ch), pl.ds(0, COORD_DIM)], sema)
            gb.wait()
            wb = pltpu.async_copy(
                rowsb_v, x_hbm.at[pl.ds(off, ch), pl.ds(COORD_DIM, COORD_DIM)],
                semb)
            wa.wait()
            wb.wait()
            return carry

        lax.fori_loop(0, n_ch, body, 0)

    return sc_gather


def _mlp_body(x, f, nf, w0ab, w0c, b0, w1, b1, w2, b2, w3, b3, out):
    bf = jnp.bfloat16
    dot = functools.partial(jnp.dot, preferred_element_type=jnp.float32)
    h = dot(x[...].astype(bf), w0ab[...]) + dot(f[...], w0c[...])
    h = jnp.maximum(h + b0[...], 0.0).astype(bf)
    h = jnp.maximum(dot(h, w1[...]) + b1[...], 0.0).astype(bf)
    h = jnp.maximum(dot(h, w2[...]) + b2[...], 0.0).astype(bf)
    out[...] = (dot(h, w3[...]) + b3[...]) * nf[...]


def _mlp_call(x, feat, nf, w0ab, w0c, b0, w1, b1, w2, b2, w3, b3):
    ec = x.shape[0]
    blk = 6400
    grid = (ec // blk,)

    def row_spec(d):
        return pl.BlockSpec((blk, d), lambda i: (i, 0))

    def full_spec(shape):
        return pl.BlockSpec(shape, lambda i: (0,) * len(shape))

    return pl.pallas_call(
        _mlp_body,
        grid=grid,
        in_specs=[
            row_spec(2 * COORD_DIM),
            row_spec(D_EDGE),
            row_spec(1),
            full_spec(w0ab.shape),
            full_spec(w0c.shape),
            full_spec(b0.shape),
            full_spec(w1.shape),
            full_spec(b1.shape),
            full_spec(w2.shape),
            full_spec(b2.shape),
            full_spec(w3.shape),
            full_spec(b3.shape),
        ],
        out_specs=pl.BlockSpec((blk, OUT_DIM), lambda i: (i, 0)),
        out_shape=jax.ShapeDtypeStruct((ec, OUT_DIM), jnp.float32),
    )(x, feat, nf, w0ab, w0c, b0, w1, b1, w2, b2, w3, b3)


N_CHUNKS = 1


def kernel(coordinates, feature_array, non_fictitious, src, dst,
           W0, b0, W1, b1, W2, b2, W3, b3):
    bf = jnp.bfloat16
    w0ab = W0[:2 * COORD_DIM].astype(bf)
    w0c = W0[2 * COORD_DIM:].astype(bf)
    feat = feature_array.astype(bf)
    nf = non_fictitious.reshape(E, 1)
    weights = (w0ab, w0c, b0.reshape(1, H),
               W1.astype(bf), b1.reshape(1, H),
               W2.astype(bf), b2.reshape(1, H),
               W3.astype(bf), b3.reshape(1, OUT_DIM))
    ec = E // N_CHUNKS
    gather = _make_sc_gather(ec)
    outs = []
    for k in range(N_CHUNKS):
        sl = slice(k * ec, (k + 1) * ec)
        x = gather(coordinates, src[sl], dst[sl])
        outs.append(_mlp_call(x, feat[sl], nf[sl], *weights))
    return jnp.concatenate(outs, axis=0)


# speedup vs baseline: 1.1860x; 1.0076x over previous
import functools

import jax
import jax.numpy as jnp
from jax import lax
from jax.experimental import pallas as pl
from jax.experimental.pallas import tpu as pltpu
from jax.experimental.pallas import tpu_sc as plsc

N_NODES = 10000
E = 320000
COORD_DIM = 128
D_EDGE = 16
H = 512
OUT_DIM = 3


@functools.cache
def _make_sc_gather(ec):
    info = plsc.get_sparse_core_info()
    nw = info.num_cores * info.num_subcores
    per_w = ec // nw
    ch = 200
    n_pairs = per_w // (2 * ch)
    mesh = plsc.VectorSubcoreMesh(core_axis_name="c", subcore_axis_name="s")

    @functools.partial(
        pl.kernel,
        mesh=mesh,
        out_type=jax.ShapeDtypeStruct((ec, 2 * COORD_DIM), jnp.float32),
        scratch_types=[
            pltpu.VMEM((ch,), jnp.int32),
            pltpu.VMEM((ch,), jnp.int32),
            pltpu.VMEM((ch,), jnp.int32),
            pltpu.VMEM((ch,), jnp.int32),
            pltpu.VMEM((ch, COORD_DIM), jnp.float32),
            pltpu.VMEM((ch, COORD_DIM), jnp.float32),
            pltpu.VMEM((ch, COORD_DIM), jnp.float32),
            pltpu.VMEM((ch, COORD_DIM), jnp.float32),
            pltpu.SemaphoreType.DMA,
            pltpu.SemaphoreType.DMA,
            pltpu.SemaphoreType.DMA,
            pltpu.SemaphoreType.DMA,
        ],
    )
    def sc_gather(coord_hbm, src_hbm, dst_hbm, x_hbm,
                  is0, id0, is1, id1, rs0, rd0, rs1, rd1, g0, g1, w0, w1):
        wid = lax.axis_index("s") * info.num_cores + lax.axis_index("c")
        base = wid * per_w

        def idx_load(c, isb, idb):
            off = base + c * ch
            pltpu.sync_copy(src_hbm.at[pl.ds(off, ch)], isb)
            pltpu.sync_copy(dst_hbm.at[pl.ds(off, ch)], idb)

        def gather_start(isb, idb, rsb, rdb, sem):
            a = pltpu.async_copy(coord_hbm.at[isb], rsb, sem)
            b = pltpu.async_copy(coord_hbm.at[idb], rdb, sem)
            return a, b

        def write_start(c, rsb, rdb, sem):
            off = base + c * ch
            a = pltpu.async_copy(
                rsb, x_hbm.at[pl.ds(off, ch), pl.ds(0, COORD_DIM)], sem)
            b = pltpu.async_copy(
                rdb, x_hbm.at[pl.ds(off, ch), pl.ds(COORD_DIM, COORD_DIM)], sem)
            return a, b

        idx_load(0, is0, id0)
        ga0, gb0 = gather_start(is0, id0, rs0, rd0, g0)
        idx_load(1, is1, id1)
        ga1, gb1 = gather_start(is1, id1, rs1, rd1, g1)

        def body(g, carry):
            c0 = 2 * g
            ga0.wait()
            gb0.wait()
            wa0, wb0 = write_start(c0, rs0, rd0, w0)
            ga1.wait()
            gb1.wait()
            wa1, wb1 = write_start(c0 + 1, rs1, rd1, w1)

            @pl.when(g + 1 < n_pairs)
            def _():
                idx_load(c0 + 2, is0, id0)
                wa0.wait()
                wb0.wait()
                gather_start(is0, id0, rs0, rd0, g0)
                idx_load(c0 + 3, is1, id1)
                wa1.wait()
                wb1.wait()
                gather_start(is1, id1, rs1, rd1, g1)

            @pl.when(g + 1 == n_pairs)
            def _():
                wa0.wait()
                wb0.wait()
                wa1.wait()
                wb1.wait()

            return carry

        lax.fori_loop(0, n_pairs, body, 0)

    return sc_gather


def _mlp_body(x, f, nf, w0ab, w0c, b0, w1, b1, w2, b2, w3, b3, out):
    bf = jnp.bfloat16
    dot = functools.partial(jnp.dot, preferred_element_type=jnp.float32)
    h = dot(x[...].astype(bf), w0ab[...]) + dot(f[...], w0c[...])
    h = jnp.maximum(h + b0[...], 0.0).astype(bf)
    h = jnp.maximum(dot(h, w1[...]) + b1[...], 0.0).astype(bf)
    h = jnp.maximum(dot(h, w2[...]) + b2[...], 0.0).astype(bf)
    out[...] = (dot(h, w3[...]) + b3[...]) * nf[...]


def _mlp_call(x, feat, nf, w0ab, w0c, b0, w1, b1, w2, b2, w3, b3):
    ec = x.shape[0]
    blk = 6400
    grid = (ec // blk,)

    def row_spec(d):
        return pl.BlockSpec((blk, d), lambda i: (i, 0))

    def full_spec(shape):
        return pl.BlockSpec(shape, lambda i: (0,) * len(shape))

    return pl.pallas_call(
        _mlp_body,
        grid=grid,
        in_specs=[
            row_spec(2 * COORD_DIM),
            row_spec(D_EDGE),
            row_spec(1),
            full_spec(w0ab.shape),
            full_spec(w0c.shape),
            full_spec(b0.shape),
            full_spec(w1.shape),
            full_spec(b1.shape),
            full_spec(w2.shape),
            full_spec(b2.shape),
            full_spec(w3.shape),
            full_spec(b3.shape),
        ],
        out_specs=pl.BlockSpec((blk, OUT_DIM), lambda i: (i, 0)),
        out_shape=jax.ShapeDtypeStruct((ec, OUT_DIM), jnp.float32),
    )(x, feat, nf, w0ab, w0c, b0, w1, b1, w2, b2, w3, b3)


N_CHUNKS = 1


def kernel(coordinates, feature_array, non_fictitious, src, dst,
           W0, b0, W1, b1, W2, b2, W3, b3):
    bf = jnp.bfloat16
    w0ab = W0[:2 * COORD_DIM].astype(bf)
    w0c = W0[2 * COORD_DIM:].astype(bf)
    feat = feature_array.astype(bf)
    nf = non_fictitious.reshape(E, 1)
    weights = (w0ab, w0c, b0.reshape(1, H),
               W1.astype(bf), b1.reshape(1, H),
               W2.astype(bf), b2.reshape(1, H),
               W3.astype(bf), b3.reshape(1, OUT_DIM))
    ec = E // N_CHUNKS
    gather = _make_sc_gather(ec)
    outs = []
    for k in range(N_CHUNKS):
        sl = slice(k * ec, (k + 1) * ec)
        x = gather(coordinates, src[sl], dst[sl])
        outs.append(_mlp_call(x, feat[sl], nf[sl], *weights))
    return jnp.concatenate(outs, axis=0)
